# baseline (speedup 1.0000x reference)
import functools

import jax
import jax.numpy as jnp
from jax import lax
from jax.experimental import pallas as pl
from jax.experimental.pallas import tpu as pltpu
from jax.experimental.pallas import tpu_sc as plsc

NUM_EMB = 100000
DIM = 128
B_TOTAL = 4096 * 50
NC, NS = 2, 16
NW = NC * NS
RPC = 64
CPW = B_TOTAL // (NW * RPC)
NBUF = 10
PREF = 7
NORM_BLK = 2000


def _tc_norm_body(w_ref, o_ref):
    w = w_ref[...]
    o_ref[...] = w * lax.rsqrt(
        jnp.mean(w * w, axis=-1, keepdims=True) + 1e-4
    )


_tc_norm = pl.pallas_call(
    _tc_norm_body,
    grid=(NUM_EMB // NORM_BLK,),
    in_specs=[pl.BlockSpec((NORM_BLK, DIM), lambda i: (i, 0))],
    out_specs=pl.BlockSpec((NORM_BLK, DIM), lambda i: (i, 0)),
    out_shape=jax.ShapeDtypeStruct((NUM_EMB, DIM), jnp.float32),
)

_mesh = plsc.VectorSubcoreMesh(core_axis_name="c", subcore_axis_name="s")


@functools.partial(
    pl.kernel,
    mesh=_mesh,
    out_type=jax.ShapeDtypeStruct((B_TOTAL, DIM), jnp.float32),
    scratch_types=[
        pltpu.VMEM((1, CPW, RPC), jnp.int32),
        pltpu.VMEM((NBUF, RPC, DIM), jnp.float32),
        pltpu.SemaphoreType.DMA((NBUF,)),
        pltpu.SemaphoreType.DMA((NBUF,)),
    ],
    compiler_params=pltpu.CompilerParams(needs_layout_passes=False),
)
def _gather(x_hbm, tab_hbm, out_hbm, idx_v, rows_v, gsem, osem):
    wid = lax.axis_index("s") * NC + lax.axis_index("c")
    out_base = wid * CPW * RPC
    pltpu.sync_copy(x_hbm.at[pl.ds(wid, 1)], idx_v)

    def start_gather(ci, b):
        pltpu.async_copy(tab_hbm.at[idx_v.at[0, ci]], rows_v.at[b], gsem.at[b])

    def wait_gather(ci, b):
        pltpu.make_async_copy(
            tab_hbm.at[idx_v.at[0, ci]], rows_v.at[b], gsem.at[b]
        ).wait()

    def out_slice(ci):
        return out_hbm.at[pl.ds(out_base + ci * RPC, RPC)]

    for b in range(PREF):
        start_gather(b, b)

    def outer(o, carry):
        for b in range(NBUF):
            ci = o * NBUF + b
            wait_gather(ci, b)
            pltpu.async_copy(rows_v.at[b], out_slice(ci), osem.at[b])
            cip = ci + PREF
            bp = (b + PREF) % NBUF

            @pl.when(cip < CPW)
            def _():
                @pl.when(cip >= NBUF)
                def _():
                    pltpu.make_async_copy(
                        rows_v.at[bp], out_slice(cip), osem.at[bp]
                    ).wait()

                start_gather(cip, bp)

        return carry

    lax.fori_loop(0, CPW // NBUF, outer, 0)
    for b in range(NBUF):
        pltpu.make_async_copy(rows_v.at[b], out_slice(b), osem.at[b]).wait()


def kernel(x, weight):
    normed = _tc_norm(weight)
    x2 = x.astype(jnp.int32).reshape(NW, CPW, RPC)
    out = _gather(x2, normed)
    return out.reshape(4096, 50, DIM)


# speedup vs baseline: 1.5863x; 1.5863x over previous
import functools

import jax
import jax.numpy as jnp
from jax import lax
from jax.experimental import pallas as pl
from jax.experimental.pallas import tpu as pltpu
from jax.experimental.pallas import tpu_sc as plsc

NUM_EMB = 100000
DIM = 128
NBATCH = 4096
SEQ = 50
NC, NS = 2, 16
NW = NC * NS
CPW = NBATCH // NW
NBUF = 8
PREF = 5
NORM_BLK = 2000


def _tc_norm_body(w_ref, o_ref):
    w = w_ref[...]
    o_ref[...] = w * lax.rsqrt(
        jnp.mean(w * w, axis=-1, keepdims=True) + 1e-4
    )


_tc_norm = pl.pallas_call(
    _tc_norm_body,
    grid=(NUM_EMB // NORM_BLK,),
    in_specs=[pl.BlockSpec((NORM_BLK, DIM), lambda i: (i, 0))],
    out_specs=pl.BlockSpec((NORM_BLK, DIM), lambda i: (i, 0)),
    out_shape=jax.ShapeDtypeStruct((NUM_EMB, DIM), jnp.float32),
)

_mesh = plsc.VectorSubcoreMesh(core_axis_name="c", subcore_axis_name="s")


@functools.partial(
    pl.kernel,
    mesh=_mesh,
    out_type=jax.ShapeDtypeStruct((NBATCH, SEQ, DIM), jnp.float32),
    scratch_types=[
        pltpu.VMEM((1, CPW, SEQ), jnp.int32),
        pltpu.VMEM((NBUF, SEQ, DIM), jnp.float32),
        pltpu.SemaphoreType.DMA((NBUF,)),
        pltpu.SemaphoreType.DMA((NBUF,)),
    ],
    compiler_params=pltpu.CompilerParams(needs_layout_passes=False),
)
def _gather(x_hbm, tab_hbm, out_hbm, idx_v, rows_v, gsem, osem):
    wid = lax.axis_index("s") * NC + lax.axis_index("c")
    out_base = wid * CPW
    pltpu.sync_copy(x_hbm.at[pl.ds(wid, 1)], idx_v)

    def start_gather(ci, b):
        pltpu.async_copy(tab_hbm.at[idx_v.at[0, ci]], rows_v.at[b], gsem.at[b])

    def wait_gather(ci, b):
        pltpu.make_async_copy(
            tab_hbm.at[idx_v.at[0, ci]], rows_v.at[b], gsem.at[b]
        ).wait()

    def out_slice(ci):
        return out_hbm.at[out_base + ci]

    for b in range(PREF):
        start_gather(b, b)

    def outer(o, carry):
        for b in range(NBUF):
            ci = o * NBUF + b
            wait_gather(ci, b)
            pltpu.async_copy(rows_v.at[b], out_slice(ci), osem.at[b])
            cip = ci + PREF
            bp = (b + PREF) % NBUF

            @pl.when(cip < CPW)
            def _():
                @pl.when(cip >= NBUF)
                def _():
                    pltpu.make_async_copy(
                        rows_v.at[bp], out_slice(cip), osem.at[bp]
                    ).wait()

                start_gather(cip, bp)

        return carry

    lax.fori_loop(0, CPW // NBUF, outer, 0)
    for b in range(NBUF):
        pltpu.make_async_copy(rows_v.at[b], out_slice(b), osem.at[b]).wait()


def kernel(x, weight):
    normed = _tc_norm(weight)
    x2 = x.astype(jnp.int32).reshape(NW, CPW, SEQ)
    return _gather(x2, normed)


# speedup vs baseline: 2.3714x; 1.4949x over previous
import functools

import jax
import jax.numpy as jnp
from jax import lax
from jax.experimental import pallas as pl
from jax.experimental.pallas import tpu as pltpu
from jax.experimental.pallas import tpu_sc as plsc

NUM_EMB = 100000
DIM = 128
NBATCH = 4096
SEQ = 50
NC, NS = 2, 16
NW = NC * NS
BPW = NBATCH // NW
CPW = SEQ
NBUF = 5
PREF = 3
NORM_BLK = 2000


def _tc_norm_body(w_ref, o_ref):
    w = w_ref[...]
    o_ref[...] = w * lax.rsqrt(
        jnp.mean(w * w, axis=-1, keepdims=True) + 1e-4
    )


_tc_norm = pl.pallas_call(
    _tc_norm_body,
    grid=(NUM_EMB // NORM_BLK,),
    in_specs=[pl.BlockSpec((NORM_BLK, DIM), lambda i: (i, 0))],
    out_specs=pl.BlockSpec((NORM_BLK, DIM), lambda i: (i, 0)),
    out_shape=jax.ShapeDtypeStruct((NUM_EMB, DIM), jnp.float32),
)

_mesh = plsc.VectorSubcoreMesh(core_axis_name="c", subcore_axis_name="s")


@functools.partial(
    pl.kernel,
    mesh=_mesh,
    out_type=jax.ShapeDtypeStruct((SEQ, NBATCH, DIM), jnp.float32),
    scratch_types=[
        pltpu.VMEM((CPW, 1, BPW), jnp.int32),
        pltpu.VMEM((NBUF, BPW, DIM), jnp.float32),
        pltpu.SemaphoreType.DMA((NBUF,)),
        pltpu.SemaphoreType.DMA((NBUF,)),
    ],
    compiler_params=pltpu.CompilerParams(needs_layout_passes=False),
)
def _gather(xt_hbm, tab_hbm, out_hbm, idx_v, rows_v, gsem, osem):
    wid = lax.axis_index("s") * NC + lax.axis_index("c")
    b0 = wid * BPW
    pltpu.sync_copy(xt_hbm.at[:, pl.ds(wid, 1)], idx_v)

    def start_gather(ci, b):
        pltpu.async_copy(tab_hbm.at[idx_v.at[ci, 0]], rows_v.at[b], gsem.at[b])

    def wait_gather(ci, b):
        pltpu.make_async_copy(
            tab_hbm.at[idx_v.at[ci, 0]], rows_v.at[b], gsem.at[b]
        ).wait()

    def out_slice(ci):
        return out_hbm.at[ci, pl.ds(b0, BPW)]

    for b in range(PREF):
        start_gather(b, b)

    def outer(o, carry):
        for b in range(NBUF):
            ci = o * NBUF + b
            wait_gather(ci, b)
            pltpu.async_copy(rows_v.at[b], out_slice(ci), osem.at[b])
            cip = ci + PREF
            bp = (b + PREF) % NBUF

            @pl.when(cip < CPW)
            def _():
                @pl.when(cip >= NBUF)
                def _():
                    pltpu.make_async_copy(
                        rows_v.at[bp], out_slice(cip), osem.at[bp]
                    ).wait()

                start_gather(cip, bp)

        return carry

    lax.fori_loop(0, CPW // NBUF, outer, 0)
    for b in range(NBUF):
        pltpu.make_async_copy(rows_v.at[b], out_slice(b), osem.at[b]).wait()


def kernel(x, weight):
    normed = _tc_norm(weight)
    xt = jnp.transpose(x.astype(jnp.int32)).reshape(SEQ, NW, BPW)
    out = _gather(xt, normed)
    return jnp.transpose(out, (1, 0, 2))


# speedup vs baseline: 2.7047x; 1.1405x over previous
import functools

import jax
import jax.numpy as jnp
from jax import lax
from jax.experimental import pallas as pl
from jax.experimental.pallas import tpu as pltpu
from jax.experimental.pallas import tpu_sc as plsc

NUM_EMB = 100000
DIM = 128
NBATCH = 4096
SEQ = 50
NC, NS = 2, 16
NW = NC * NS
BPW = NBATCH // NW
CPW = SEQ
NBUF = 5
PREF = 3
NORM_BLK = 5000


def _tc_norm_body(w_ref, o_ref):
    w = w_ref[...]
    o_ref[...] = w * lax.rsqrt(
        jnp.mean(w * w, axis=-1, keepdims=True) + 1e-4
    )


_tc_norm = pl.pallas_call(
    _tc_norm_body,
    grid=(NUM_EMB // NORM_BLK,),
    in_specs=[pl.BlockSpec((NORM_BLK, DIM), lambda i: (i, 0))],
    out_specs=pl.BlockSpec((NORM_BLK, DIM), lambda i: (i, 0)),
    out_shape=jax.ShapeDtypeStruct((NUM_EMB, DIM), jnp.float32),
)

_mesh = plsc.VectorSubcoreMesh(core_axis_name="c", subcore_axis_name="s")


@functools.partial(
    pl.kernel,
    mesh=_mesh,
    out_type=jax.ShapeDtypeStruct((SEQ, NBATCH, DIM), jnp.float32),
    scratch_types=[
        pltpu.VMEM((CPW, 1, BPW), jnp.int32),
        pltpu.VMEM((NBUF, BPW, DIM), jnp.float32),
        pltpu.SemaphoreType.DMA((NBUF,)),
        pltpu.SemaphoreType.DMA((NBUF,)),
    ],
    compiler_params=pltpu.CompilerParams(needs_layout_passes=False),
)
def _gather(xt_hbm, tab_hbm, out_hbm, idx_v, rows_v, gsem, osem):
    wid = lax.axis_index("s") * NC + lax.axis_index("c")
    b0 = wid * BPW
    pltpu.sync_copy(xt_hbm.at[:, pl.ds(wid, 1)], idx_v)

    def start_gather(ci, b):
        pltpu.async_copy(tab_hbm.at[idx_v.at[ci, 0]], rows_v.at[b], gsem.at[b])

    def wait_gather(ci, b):
        pltpu.make_async_copy(
            tab_hbm.at[idx_v.at[ci, 0]], rows_v.at[b], gsem.at[b]
        ).wait()

    def out_slice(ci):
        return out_hbm.at[ci, pl.ds(b0, BPW)]

    for b in range(PREF):
        start_gather(b, b)

    def outer(o, carry):
        for b in range(NBUF):
            ci = o * NBUF + b
            wait_gather(ci, b)
            pltpu.async_copy(rows_v.at[b], out_slice(ci), osem.at[b])
            cip = ci + PREF
            bp = (b + PREF) % NBUF

            @pl.when(cip < CPW)
            def _():
                @pl.when(cip >= NBUF)
                def _():
                    pltpu.make_async_copy(
                        rows_v.at[bp], out_slice(cip), osem.at[bp]
                    ).wait()

                start_gather(cip, bp)

        return carry

    lax.fori_loop(0, CPW // NBUF, outer, 0)
    for b in range(NBUF):
        pltpu.make_async_copy(rows_v.at[b], out_slice(b), osem.at[b]).wait()


def kernel(x, weight):
    normed = _tc_norm(weight)
    xt = jnp.transpose(x.astype(jnp.int32)).reshape(SEQ, NW, BPW)
    out = _gather(xt, normed)
    return jnp.transpose(out, (1, 0, 2))


# speedup vs baseline: 3.4699x; 1.2829x over previous
import functools

import jax
import jax.numpy as jnp
from jax import lax
from jax.experimental import pallas as pl
from jax.experimental.pallas import tpu as pltpu
from jax.experimental.pallas import tpu_sc as plsc

NUM_EMB = 100000
DIM = 128
NBATCH = 4096
SEQ = 50
NC, NS = 2, 16
NW = NC * NS
BPW = NBATCH // NW
CPW = SEQ
NBUF = 5
PREF = 3
NORM_BLK = 5000


def _tc_norm_body(w_ref, o_ref):
    w = w_ref[...]
    o_ref[...] = w * lax.rsqrt(
        jnp.mean(w * w, axis=-1, keepdims=True) + 1e-4
    )


_tc_norm = pl.pallas_call(
    _tc_norm_body,
    grid=(NUM_EMB // NORM_BLK,),
    in_specs=[pl.BlockSpec((NORM_BLK, DIM), lambda i: (i, 0))],
    out_specs=pl.BlockSpec((NORM_BLK, DIM), lambda i: (i, 0)),
    out_shape=jax.ShapeDtypeStruct((NUM_EMB, DIM), jnp.float32),
)

_mesh = plsc.VectorSubcoreMesh(core_axis_name="c", subcore_axis_name="s")


@functools.partial(
    pl.kernel,
    mesh=_mesh,
    out_type=jax.ShapeDtypeStruct((SEQ, NBATCH, DIM), jnp.float32),
    scratch_types=[
        pltpu.VMEM((CPW, 1, BPW), jnp.int32),
        pltpu.VMEM((NBUF, BPW, DIM), jnp.float32),
        pltpu.SemaphoreType.DMA((NBUF,)),
        pltpu.SemaphoreType.DMA((NBUF,)),
    ],
    compiler_params=pltpu.CompilerParams(needs_layout_passes=False),
)
def _gather(xt_hbm, tab_hbm, out_hbm, idx_v, rows_v, gsem, osem):
    wid = lax.axis_index("s") * NC + lax.axis_index("c")
    b0 = wid * BPW

    def _rsqrt_nr(v):
        i = lax.bitcast_convert_type(v, jnp.int32)
        i = jnp.int32(0x5F3759DF) - lax.shift_right_arithmetic(i, 1)
        y = lax.bitcast_convert_type(i, jnp.float32)
        for _ in range(3):
            y = y * (1.5 - 0.5 * v * y * y)
        return y

    def _norm_row(rows, r):
        vs = [rows[r, pl.ds(k * 16, 16)] for k in range(DIM // 16)]
        acc = vs[0] * vs[0]
        for v in vs[1:]:
            acc = acc + v * v
        s = jnp.sum(acc)
        scale = _rsqrt_nr(jnp.full((16,), s * (1.0 / DIM) + 1e-4, jnp.float32))
        for k, v in enumerate(vs):
            rows[r, pl.ds(k * 16, 16)] = v * scale

    def compute(b):
        rows = rows_v.at[b]

        def rows_body(i, c):
            for u in range(8):
                _norm_row(rows, i * 8 + u)
            return c

        lax.fori_loop(0, BPW // 8, rows_body, 0)
    pltpu.sync_copy(xt_hbm.at[:, pl.ds(wid, 1)], idx_v)

    def start_gather(ci, b):
        pltpu.async_copy(tab_hbm.at[idx_v.at[ci, 0]], rows_v.at[b], gsem.at[b])

    def wait_gather(ci, b):
        pltpu.make_async_copy(
            tab_hbm.at[idx_v.at[ci, 0]], rows_v.at[b], gsem.at[b]
        ).wait()

    def out_slice(ci):
        return out_hbm.at[ci, pl.ds(b0, BPW)]

    for b in range(PREF):
        start_gather(b, b)

    def outer(o, carry):
        for b in range(NBUF):
            ci = o * NBUF + b
            wait_gather(ci, b)
            compute(b)
            pltpu.async_copy(rows_v.at[b], out_slice(ci), osem.at[b])
            cip = ci + PREF
            bp = (b + PREF) % NBUF

            @pl.when(cip < CPW)
            def _():
                @pl.when(cip >= NBUF)
                def _():
                    pltpu.make_async_copy(
                        rows_v.at[bp], out_slice(cip), osem.at[bp]
                    ).wait()

                start_gather(cip, bp)

        return carry

    lax.fori_loop(0, CPW // NBUF, outer, 0)
    for b in range(NBUF):
        pltpu.make_async_copy(rows_v.at[b], out_slice(b), osem.at[b]).wait()


def kernel(x, weight):
    xt = jnp.transpose(x.astype(jnp.int32)).reshape(SEQ, NW, BPW)
    out = _gather(xt, weight)
    return jnp.transpose(out, (1, 0, 2))


# speedup vs baseline: 3.5912x; 1.0349x over previous
import functools

import jax
import jax.numpy as jnp
from jax import lax
from jax.experimental import pallas as pl
from jax.experimental.pallas import tpu as pltpu
from jax.experimental.pallas import tpu_sc as plsc

NUM_EMB = 100000
DIM = 128
NBATCH = 4096
SEQ = 50
NC, NS = 2, 16
NW = NC * NS
BPW = NBATCH // NW
CPW = SEQ
NBUF = 5
PREF = 3
NORM_BLK = 5000


def _tc_norm_body(w_ref, o_ref):
    w = w_ref[...]
    o_ref[...] = w * lax.rsqrt(
        jnp.mean(w * w, axis=-1, keepdims=True) + 1e-4
    )


_tc_norm = pl.pallas_call(
    _tc_norm_body,
    grid=(NUM_EMB // NORM_BLK,),
    in_specs=[pl.BlockSpec((NORM_BLK, DIM), lambda i: (i, 0))],
    out_specs=pl.BlockSpec((NORM_BLK, DIM), lambda i: (i, 0)),
    out_shape=jax.ShapeDtypeStruct((NUM_EMB, DIM), jnp.float32),
)

_mesh = plsc.VectorSubcoreMesh(core_axis_name="c", subcore_axis_name="s")


@functools.partial(
    pl.kernel,
    mesh=_mesh,
    out_type=jax.ShapeDtypeStruct((SEQ, NBATCH, DIM), jnp.float32),
    scratch_types=[
        pltpu.VMEM((CPW, 1, BPW), jnp.int32),
        pltpu.VMEM((NBUF, BPW, DIM), jnp.float32),
        pltpu.SemaphoreType.DMA((NBUF,)),
        pltpu.SemaphoreType.DMA((NBUF,)),
    ],
    compiler_params=pltpu.CompilerParams(needs_layout_passes=False),
)
def _gather(xt_hbm, tab_hbm, out_hbm, idx_v, rows_v, gsem, osem):
    wid = lax.axis_index("s") * NC + lax.axis_index("c")
    b0 = wid * BPW

    def _rsqrt_nr(v):
        i = lax.bitcast_convert_type(v, jnp.int32)
        i = jnp.int32(0x5F3759DF) - lax.shift_right_arithmetic(i, 1)
        y = lax.bitcast_convert_type(i, jnp.float32)
        for _ in range(3):
            y = y * (1.5 - 0.5 * v * y * y)
        return y

    def _norm_row(rows, r):
        vs = [rows[r, pl.ds(k * 16, 16)] for k in range(DIM // 16)]
        acc = vs[0] * vs[0]
        for v in vs[1:]:
            acc = acc + v * v
        s = jnp.sum(acc)
        scale = _rsqrt_nr(jnp.full((16,), s * (1.0 / DIM) + 1e-4, jnp.float32))
        for k, v in enumerate(vs):
            rows[r, pl.ds(k * 16, 16)] = v * scale

    def compute(b):
        rows = rows_v.at[b]

        def rows_body(i, c):
            for u in range(16):
                _norm_row(rows, i * 16 + u)
            return c

        lax.fori_loop(0, BPW // 16, rows_body, 0)
    pltpu.sync_copy(xt_hbm.at[:, pl.ds(wid, 1)], idx_v)

    def start_gather(ci, b):
        pltpu.async_copy(tab_hbm.at[idx_v.at[ci, 0]], rows_v.at[b], gsem.at[b])

    def wait_gather(ci, b):
        pltpu.make_async_copy(
            tab_hbm.at[idx_v.at[ci, 0]], rows_v.at[b], gsem.at[b]
        ).wait()

    def out_slice(ci):
        return out_hbm.at[ci, pl.ds(b0, BPW)]

    for b in range(PREF):
        start_gather(b, b)

    def outer(o, carry):
        for b in range(NBUF):
            ci = o * NBUF + b
            wait_gather(ci, b)
            compute(b)
            pltpu.async_copy(rows_v.at[b], out_slice(ci), osem.at[b])
            cip = ci + PREF
            bp = (b + PREF) % NBUF

            @pl.when(cip < CPW)
            def _():
                @pl.when(cip >= NBUF)
                def _():
                    pltpu.make_async_copy(
                        rows_v.at[bp], out_slice(cip), osem.at[bp]
                    ).wait()

                start_gather(cip, bp)

        return carry

    lax.fori_loop(0, CPW // NBUF, outer, 0)
    for b in range(NBUF):
        pltpu.make_async_copy(rows_v.at[b], out_slice(b), osem.at[b]).wait()


def kernel(x, weight):
    xt = jnp.transpose(x.astype(jnp.int32)).reshape(SEQ, NW, BPW)
    out = _gather(xt, weight)
    return jnp.transpose(out, (1, 0, 2))


# speedup vs baseline: 3.5922x; 1.0003x over previous
import functools

import jax
import jax.numpy as jnp
from jax import lax
from jax.experimental import pallas as pl
from jax.experimental.pallas import tpu as pltpu
from jax.experimental.pallas import tpu_sc as plsc

NUM_EMB = 100000
DIM = 128
NBATCH = 4096
SEQ = 50
NC, NS = 2, 16
NW = NC * NS
BPW = NBATCH // NW
CPW = SEQ
NBUF = 5
PREF = 3

_mesh = plsc.VectorSubcoreMesh(core_axis_name="c", subcore_axis_name="s")


@functools.partial(
    pl.kernel,
    mesh=_mesh,
    out_type=jax.ShapeDtypeStruct((SEQ, NBATCH, DIM), jnp.float32),
    scratch_types=[
        pltpu.VMEM((CPW, 1, BPW), jnp.int32),
        pltpu.VMEM((NBUF, BPW, DIM), jnp.float32),
        pltpu.SemaphoreType.DMA((NBUF,)),
        pltpu.SemaphoreType.DMA((NBUF,)),
    ],
    compiler_params=pltpu.CompilerParams(needs_layout_passes=False),
)
def _embed(xt_hbm, tab_hbm, out_hbm, idx_v, rows_v, gsem, osem):
    wid = lax.axis_index("s") * NC + lax.axis_index("c")
    b0 = wid * BPW

    def _rsqrt_nr(v):
        i = lax.bitcast_convert_type(v, jnp.int32)
        i = jnp.int32(0x5F3759DF) - lax.shift_right_arithmetic(i, 1)
        y = lax.bitcast_convert_type(i, jnp.float32)
        for _ in range(3):
            y = y * (1.5 - 0.5 * v * y * y)
        return y

    def _norm_row(rows, r):
        vs = [rows[r, pl.ds(k * 16, 16)] for k in range(DIM // 16)]
        acc = vs[0] * vs[0]
        for v in vs[1:]:
            acc = acc + v * v
        s = jnp.sum(acc)
        scale = _rsqrt_nr(jnp.full((16,), s * (1.0 / DIM) + 1e-4, jnp.float32))
        for k, v in enumerate(vs):
            rows[r, pl.ds(k * 16, 16)] = v * scale

    def compute(b):
        rows = rows_v.at[b]

        def rows_body(i, c):
            for u in range(16):
                _norm_row(rows, i * 16 + u)
            return c

        lax.fori_loop(0, BPW // 16, rows_body, 0)

    pltpu.sync_copy(xt_hbm.at[:, pl.ds(wid, 1)], idx_v)

    def start_gather(ci, b):
        pltpu.async_copy(tab_hbm.at[idx_v.at[ci, 0]], rows_v.at[b], gsem.at[b])

    def wait_gather(ci, b):
        pltpu.make_async_copy(
            tab_hbm.at[idx_v.at[ci, 0]], rows_v.at[b], gsem.at[b]
        ).wait()

    def out_slice(ci):
        return out_hbm.at[ci, pl.ds(b0, BPW)]

    for b in range(PREF):
        start_gather(b, b)

    def outer(o, carry):
        for b in range(NBUF):
            ci = o * NBUF + b
            wait_gather(ci, b)
            compute(b)
            pltpu.async_copy(rows_v.at[b], out_slice(ci), osem.at[b])
            cip = ci + PREF
            bp = (b + PREF) % NBUF

            @pl.when(cip < CPW)
            def _():
                @pl.when(cip >= NBUF)
                def _():
                    pltpu.make_async_copy(
                        rows_v.at[bp], out_slice(cip), osem.at[bp]
                    ).wait()

                start_gather(cip, bp)

        return carry

    lax.fori_loop(0, CPW // NBUF, outer, 0)
    for b in range(NBUF):
        pltpu.make_async_copy(rows_v.at[b], out_slice(b), osem.at[b]).wait()


def kernel(x, weight):
    xt = jnp.transpose(x.astype(jnp.int32)).reshape(SEQ, NW, BPW)
    out = _embed(xt, weight)
    return jnp.transpose(out, (1, 0, 2))
